# baseline (speedup 1.0000x reference)
import functools

import jax
import jax.numpy as jnp
from jax import lax
from jax.experimental import pallas as pl
from jax.experimental.pallas import tpu as pltpu
from jax.experimental.pallas import tpu_sc as plsc

VOCAB = 1000000
DIM = 64
B = 16384

CW = 512
NCHUNK_FULL = VOCAB // CW
TAIL_START = NCHUNK_FULL * CW
TAIL_W = VOCAB - TAIL_START
SLOTS_PER_W = 62
RANGE_W = SLOTS_PER_W * CW
RCAP = 6144
TCAP = 256
RING = 128
SEG = 4096
SENT = 1 << 30
SLOT_SZ = 112


def _build():
    info = plsc.get_sparse_core_info()
    nc, ns = info.num_cores, info.num_subcores
    mesh = plsc.VectorSubcoreMesh(core_axis_name="c", subcore_axis_name="s")

    @functools.partial(
        pl.kernel,
        mesh=mesh,
        out_type=jax.ShapeDtypeStruct((3 * B * DIM,), jnp.float32),
        scratch_types=(
            [pltpu.VMEM((SEG,), jnp.int32)]
            + [pltpu.VMEM((64 * SLOT_SZ + 32,), jnp.int32)] * 2
            + [pltpu.VMEM((80,), jnp.int32)]
            + [pltpu.VMEM((64, CW), jnp.float32)] * 2
            + [pltpu.VMEM((RING * DIM,), jnp.float32)]
            + [pltpu.VMEM((64, TAIL_W), jnp.float32)]
            + [pltpu.SemaphoreType.DMA] * 4
        ),
        compiler_params=pltpu.CompilerParams(use_tc_tiling_on_sc=True,
                                             needs_layout_passes=False),
    )
    def triple_gather(a_hbm, p_hbm, n_hbm, tt_hbm, out_hbm,
                      seg_v, sidx, sdst, hist,
                      slab0, slab1, ring_v, tail_v,
                      sem_seg, sem0, sem1, sem_out):
        wid = lax.axis_index("s") * nc + lax.axis_index("c")
        lo = wid * RANGE_W
        hi = lo + RANGE_W
        lanes = lax.iota(jnp.int32, 16)
        zeros = jnp.full((16,), 0, jnp.int32)

        def splat_load(ref, p):
            return plsc.load_gather(ref, [zeros + p])

        def bucket_of(v):
            cc = lax.min(lax.shift_right_logical(v - lo, 9), zeros + 63)
            return jnp.where(v >= TAIL_START, zeros + 63, cc)

        for z in range(4):
            hist[pl.ds(z * 16, 16)] = (lanes + z * 16) * SLOT_SZ

        for s, src in enumerate((a_hbm, p_hbm, n_hbm)):
            for seg_i in range(B // SEG):
                pltpu.sync_copy(src.at[pl.ds(seg_i * SEG, SEG)], seg_v)
                dbase = s * B + seg_i * SEG

                def scan(g, _, dbase=dbase):
                    v = seg_v[pl.ds(g * 16, 16)]
                    main = jnp.logical_and(v >= lo, v < hi)
                    nmain = plsc.all_reduce_population_count(main)[0]

                    def match(k, m, dbase=dbase, g=g):
                        l = plsc.all_reduce_ffs(m != 0)[0]
                        p = g * 16 + l
                        iv = splat_load(seg_v, p)
                        ccv = bucket_of(iv)
                        slotv = plsc.load_gather(hist, [ccv])
                        slot = slotv[0]
                        sidx[pl.ds(slot, 16)] = iv
                        sdst[pl.ds(slot, 16)] = zeros + (dbase + p)
                        plsc.store_scatter(
                            hist, [ccv],
                            lax.min(slotv + 1, ccv * SLOT_SZ + 96),
                            mask=lanes == 0)
                        return jnp.where(lanes == l, 0, m)

                    lax.fori_loop(0, nmain, match, main.astype(jnp.int32),
                                  unroll=False)
                    return 0

                lax.fori_loop(0, SEG // 16, scan, 0, unroll=4)

        slabs = (slab0, slab1)
        sems = (sem0, sem1)

        def group_emit(idx_ref, dst_ref, o0, o1, slab_ref, start, oc):
            ngroups = lax.div(o1 - o0 + 15, jnp.int32(16))

            def g_body(g, oc):
                base = o0 + g * 16
                pos = lax.min(base + lanes, o1 - 1)
                iv = plsc.load_gather(idx_ref, [pos])
                lv = iv - start
                slotb = lax.rem(oc, jnp.int32(RING))

                @pl.when(oc >= RING)
                def _wait_group():
                    pltpu.make_async_copy(
                        out_hbm.at[pl.ds(0, 16 * DIM)],
                        ring_v.at[pl.ds(slotb * DIM, 16 * DIM)],
                        sem_out).wait()

                for f in range(DIM):
                    col = plsc.load_gather(slab_ref, [zeros + f, lv])
                    plsc.store_scatter(
                        ring_v, [(slotb + lanes) * DIM + f], col)
                for j in range(16):
                    dj = splat_load(dst_ref, lax.min(base + j, o1 - 1))[0]
                    pltpu.async_copy(
                        ring_v.at[pl.ds((slotb + j) * DIM, DIM)],
                        out_hbm.at[pl.ds(dj * DIM, DIM)], sem_out)
                return oc + 16

            return lax.fori_loop(0, ngroups, g_body, oc, unroll=False)

        def chunk_start(c_loc):
            return lax.min(lo + c_loc * CW, jnp.int32((NCHUNK_FULL - 1) * CW))

        def fire(c_loc, buf):
            pltpu.async_copy(
                tt_hbm.at[:, pl.ds(chunk_start(c_loc), CW)],
                slabs[buf], sems[buf])

        def drain(buf):
            pltpu.make_async_copy(tt_hbm.at[:, pl.ds(0, CW)],
                                  slabs[buf], sems[buf]).wait()

        def process(c_loc, buf, oc):
            start = chunk_start(c_loc)
            o0 = c_loc * SLOT_SZ
            o1 = splat_load(hist, c_loc)[0]
            return group_emit(sidx, sdst, o0, o1, slabs[buf], start, oc)

        fire(jnp.int32(0), 0)

        def pair(t, oc):
            c0 = t * 2
            fire(c0 + 1, 1)
            drain(0)
            oc = process(c0, 0, oc)
            fire(c0 + 2, 0)
            drain(1)
            oc = process(c0 + 1, 1, oc)
            return oc

        npairs = lax.div(
            lax.min(jnp.int32(SLOTS_PER_W),
                    jnp.int32(NCHUNK_FULL + 1) - wid * SLOTS_PER_W) + 1,
            jnp.int32(2))
        oc = lax.fori_loop(0, npairs, pair, jnp.int32(0), unroll=False)
        drain(0)

        pltpu.sync_copy(tt_hbm.at[:, pl.ds(TAIL_START, TAIL_W)], tail_v)
        oc = group_emit(sidx, sdst, jnp.int32(63 * SLOT_SZ),
                        splat_load(hist, 63)[0], tail_v,
                        jnp.int32(TAIL_START), oc)

        def reclaim(k, _):
            pltpu.make_async_copy(out_hbm.at[pl.ds(0, 16 * DIM)],
                                  ring_v.at[pl.ds(0, 16 * DIM)],
                                  sem_out).wait()
            return ()

        lax.fori_loop(0, lax.div(lax.min(oc, jnp.int32(RING)), jnp.int32(16)),
                      reclaim, (), unroll=False)

    return triple_gather


_TRIPLE_GATHER = _build()


@jax.jit
def kernel(anchor, pos, neg, table):
    a = anchor.astype(jnp.int32)
    p = pos.astype(jnp.int32)
    n = neg.astype(jnp.int32)
    flat = _TRIPLE_GATHER(a, p, n, table.T)
    oa = flat[0:B * DIM]
    op_ = flat[B * DIM:2 * B * DIM]
    on = flat[2 * B * DIM:3 * B * DIM]
    return (oa.reshape(-1, 1), op_.reshape(-1, 1), on.reshape(-1, 1))


# speedup vs baseline: 1.0057x; 1.0057x over previous
import functools

import jax
import jax.numpy as jnp
from jax import lax
from jax.experimental import pallas as pl
from jax.experimental.pallas import tpu as pltpu
from jax.experimental.pallas import tpu_sc as plsc

VOCAB = 1000000
DIM = 64
B = 16384

CW = 512
NCHUNK_FULL = VOCAB // CW
TAIL_START = NCHUNK_FULL * CW
TAIL_W = VOCAB - TAIL_START
SLOTS_PER_W = 62
RANGE_W = SLOTS_PER_W * CW
RCAP = 6144
TCAP = 256
RING = 128
SEG = 4096
SENT = 1 << 30
SLOT_SZ = 112


def _build():
    info = plsc.get_sparse_core_info()
    nc, ns = info.num_cores, info.num_subcores
    mesh = plsc.VectorSubcoreMesh(core_axis_name="c", subcore_axis_name="s")

    @functools.partial(
        pl.kernel,
        mesh=mesh,
        out_type=jax.ShapeDtypeStruct((3 * B * DIM,), jnp.float32),
        scratch_types=(
            [pltpu.VMEM((SEG,), jnp.int32)]
            + [pltpu.VMEM((64 * SLOT_SZ + 32,), jnp.int32)] * 2
            + [pltpu.VMEM((80,), jnp.int32)]
            + [pltpu.VMEM((64, CW), jnp.float32)] * 2
            + [pltpu.VMEM((RING * DIM,), jnp.float32)]
            + [pltpu.VMEM((64, TAIL_W), jnp.float32)]
            + [pltpu.SemaphoreType.DMA] * 4
        ),
        compiler_params=pltpu.CompilerParams(use_tc_tiling_on_sc=True,
                                             needs_layout_passes=False),
    )
    def triple_gather(a_hbm, p_hbm, n_hbm, tt_hbm, out_hbm,
                      seg_v, sidx, sdst, hist,
                      slab0, slab1, ring_v, tail_v,
                      sem_seg, sem0, sem1, sem_out):
        wid = lax.axis_index("s") * nc + lax.axis_index("c")
        lo = wid * RANGE_W
        hi = lo + RANGE_W
        lanes = lax.iota(jnp.int32, 16)
        zeros = jnp.full((16,), 0, jnp.int32)

        def splat_load(ref, p):
            return plsc.load_gather(ref, [zeros + p])

        def bucket_of(v):
            cc = lax.min(lax.shift_right_logical(v - lo, 9), zeros + 63)
            return jnp.where(v >= TAIL_START, zeros + 63, cc)

        for z in range(4):
            hist[pl.ds(z * 16, 16)] = (lanes + z * 16) * SLOT_SZ

        for s, src in enumerate((a_hbm, p_hbm, n_hbm)):
            for seg_i in range(B // SEG):
                pltpu.sync_copy(src.at[pl.ds(seg_i * SEG, SEG)], seg_v)
                dbase = s * B + seg_i * SEG

                def scan(g, _, dbase=dbase):
                    v = seg_v[pl.ds(g * 16, 16)]
                    main = jnp.logical_and(v >= lo, v < hi)
                    nmain = plsc.all_reduce_population_count(main)[0]

                    def match(k, m, dbase=dbase, g=g):
                        l = plsc.all_reduce_ffs(m != 0)[0]
                        p = g * 16 + l
                        iv = splat_load(seg_v, p)
                        ccv = bucket_of(iv)
                        slotv = plsc.load_gather(hist, [ccv])
                        slot = slotv[0]
                        sidx[pl.ds(slot, 16)] = iv
                        sdst[pl.ds(slot, 16)] = zeros + (dbase + p)
                        plsc.store_scatter(
                            hist, [ccv],
                            lax.min(slotv + 1, ccv * SLOT_SZ + 96),
                            mask=lanes == 0)
                        return jnp.where(lanes == l, 0, m)

                    lax.fori_loop(0, nmain, match, main.astype(jnp.int32),
                                  unroll=False)
                    return 0

                lax.fori_loop(0, SEG // 16, scan, 0, unroll=2)

        slabs = (slab0, slab1)
        sems = (sem0, sem1)

        def group_emit(idx_ref, dst_ref, o0, o1, slab_ref, start, oc):
            ngroups = lax.div(o1 - o0 + 15, jnp.int32(16))

            def g_body(g, oc):
                base = o0 + g * 16
                pos = lax.min(base + lanes, o1 - 1)
                iv = plsc.load_gather(idx_ref, [pos])
                lv = iv - start
                slotb = lax.rem(oc, jnp.int32(RING))

                @pl.when(oc >= RING)
                def _wait_group():
                    pltpu.make_async_copy(
                        out_hbm.at[pl.ds(0, 16 * DIM)],
                        ring_v.at[pl.ds(slotb * DIM, 16 * DIM)],
                        sem_out).wait()

                for f in range(DIM):
                    col = plsc.load_gather(slab_ref, [zeros + f, lv])
                    plsc.store_scatter(
                        ring_v, [(slotb + lanes) * DIM + f], col)
                for j in range(16):
                    dj = splat_load(dst_ref, lax.min(base + j, o1 - 1))[0]
                    pltpu.async_copy(
                        ring_v.at[pl.ds((slotb + j) * DIM, DIM)],
                        out_hbm.at[pl.ds(dj * DIM, DIM)], sem_out)
                return oc + 16

            return lax.fori_loop(0, ngroups, g_body, oc, unroll=False)

        def chunk_start(c_loc):
            return lax.min(lo + c_loc * CW, jnp.int32((NCHUNK_FULL - 1) * CW))

        def fire(c_loc, buf):
            pltpu.async_copy(
                tt_hbm.at[:, pl.ds(chunk_start(c_loc), CW)],
                slabs[buf], sems[buf])

        def drain(buf):
            pltpu.make_async_copy(tt_hbm.at[:, pl.ds(0, CW)],
                                  slabs[buf], sems[buf]).wait()

        def process(c_loc, buf, oc):
            start = chunk_start(c_loc)
            o0 = c_loc * SLOT_SZ
            o1 = splat_load(hist, c_loc)[0]
            return group_emit(sidx, sdst, o0, o1, slabs[buf], start, oc)

        fire(jnp.int32(0), 0)

        def pair(t, oc):
            c0 = t * 2
            fire(c0 + 1, 1)
            drain(0)
            oc = process(c0, 0, oc)
            fire(c0 + 2, 0)
            drain(1)
            oc = process(c0 + 1, 1, oc)
            return oc

        npairs = lax.div(
            lax.min(jnp.int32(SLOTS_PER_W),
                    jnp.int32(NCHUNK_FULL + 1) - wid * SLOTS_PER_W) + 1,
            jnp.int32(2))
        oc = lax.fori_loop(0, npairs, pair, jnp.int32(0), unroll=False)
        drain(0)

        pltpu.sync_copy(tt_hbm.at[:, pl.ds(TAIL_START, TAIL_W)], tail_v)
        oc = group_emit(sidx, sdst, jnp.int32(63 * SLOT_SZ),
                        splat_load(hist, 63)[0], tail_v,
                        jnp.int32(TAIL_START), oc)

        def reclaim(k, _):
            pltpu.make_async_copy(out_hbm.at[pl.ds(0, 16 * DIM)],
                                  ring_v.at[pl.ds(0, 16 * DIM)],
                                  sem_out).wait()
            return ()

        lax.fori_loop(0, lax.div(lax.min(oc, jnp.int32(RING)), jnp.int32(16)),
                      reclaim, (), unroll=False)

    return triple_gather


_TRIPLE_GATHER = _build()


@jax.jit
def kernel(anchor, pos, neg, table):
    a = anchor.astype(jnp.int32)
    p = pos.astype(jnp.int32)
    n = neg.astype(jnp.int32)
    flat = _TRIPLE_GATHER(a, p, n, table.T)
    oa = flat[0:B * DIM]
    op_ = flat[B * DIM:2 * B * DIM]
    on = flat[2 * B * DIM:3 * B * DIM]
    return (oa.reshape(-1, 1), op_.reshape(-1, 1), on.reshape(-1, 1))
